# Initial kernel scaffold; baseline (speedup 1.0000x reference)
import jax
import jax.numpy as jnp
from jax.experimental import pallas as pl


def kernel(magno_patches, vit_positional_embedding, scores):
    raise NotImplementedError("write your pallas kernel here")


# speedup vs baseline: 1.7505x; 1.7505x over previous
import functools

import jax
import jax.numpy as jnp
from jax import lax
from jax.experimental import pallas as pl
from jax.experimental.pallas import tpu as pltpu
from jax.experimental.pallas import tpu_sc as plsc

B, N, D = 64, 1024, 768
K = 256
NC, NS, L = 2, 16, 16
NW = NC * NS
ROWS_PER_W = B // NW
CHUNK = 64
NCHUNK = K // CHUNK


def _gather_add_kernel(magno_flat, pos_flat, idx, out,
                       idxp_v, idxq_v, topidx_v, pbuf, qbuf, sem_p, sem_q):
    wid = lax.axis_index("s") * NC + lax.axis_index("c")
    for r in range(ROWS_PER_W):
        b = wid * ROWS_PER_W + r
        pltpu.sync_copy(idx.at[b], topidx_v)
        boff = b * N
        for g in range(K // L):
            v = topidx_v[pl.ds(g * L, L)]
            idxp_v[pl.ds(g * L, L)] = v + boff
            idxq_v[pl.ds(g * L, L)] = v + 1
        for c in range(NCHUNK):
            cp = pltpu.async_copy(
                magno_flat.at[idxp_v.at[pl.ds(c * CHUNK, CHUNK)]], pbuf, sem_p)
            cq = pltpu.async_copy(
                pos_flat.at[idxq_v.at[pl.ds(c * CHUNK, CHUNK)]], qbuf, sem_q)
            cp.wait()
            cq.wait()

            def add_row(i):
                for j in range(D // L):
                    s = pl.ds(j * L, L)
                    pbuf[i, s] = pbuf[i, s] + qbuf[i, s]
            lax.fori_loop(0, CHUNK, lambda i, _: (add_row(i), 0)[1], 0)
            pltpu.sync_copy(pbuf, out.at[b, pl.ds(c * CHUNK, CHUNK), :])


@jax.jit
def _sc_gather_add(magno_flat, pos_flat, idx):
    mesh = plsc.VectorSubcoreMesh(core_axis_name="c", subcore_axis_name="s",
                                  num_cores=NC, num_subcores=NS)
    return pl.kernel(
        _gather_add_kernel,
        out_type=jax.ShapeDtypeStruct((B, K, D), jnp.float32),
        mesh=mesh,
        scratch_types=[
            pltpu.VMEM((K,), jnp.int32),
            pltpu.VMEM((K,), jnp.int32),
            pltpu.VMEM((K,), jnp.int32),
            pltpu.VMEM((CHUNK, D), jnp.float32),
            pltpu.VMEM((CHUNK, D), jnp.float32),
            pltpu.SemaphoreType.DMA,
            pltpu.SemaphoreType.DMA,
        ],
    )(magno_flat, pos_flat, idx)


def kernel(magno_patches, vit_positional_embedding, scores):
    _, idx = jax.lax.top_k(scores, K)
    magno_flat = magno_patches.reshape(B * N, D)
    pos_flat = vit_positional_embedding.reshape(N + 1, D)
    return _sc_gather_add(magno_flat, pos_flat, idx.astype(jnp.int32))
